# baseline (speedup 1.0000x reference)
import functools

import jax
import jax.numpy as jnp
import numpy as np
from jax import lax
from jax.experimental import pallas as pl
from jax.experimental.pallas import tpu as pltpu
from jax.experimental.pallas import tpu_sc as plsc

_N = 10000
_NP = 10240
_H = 128
_NNZ = 320000
_NC = 2
_NS = 16
_NW = _NC * _NS
_K = 128
_C0 = 160
_NCHUNK = _NS * _C0
_BLK = 16
_EPAD = _K * _NCHUNK
_RPS = _NP // _NS


def _swish(x):
    return x * jax.nn.sigmoid(x)


def _bcast_lane(vv, l):
    return lax.gather(
        vv, jnp.full((16, 1), l, jnp.int32),
        lax.GatherDimensionNumbers(
            offset_dims=(), collapsed_slice_dims=(0,), start_index_map=(0,)),
        (1,), mode=lax.GatherScatterMode.PROMISE_IN_BOUNDS)


def _scale(idx_v, j, buf):
    def group(g, c2):
        vv = idx_v[j, pl.ds(g * 16, 16)]
        for l in range(16):
            v = _bcast_lane(vv, l)
            e = g * 16 + l
            for r in range(_H // 16):
                sl = pl.ds(r * 16, 16)
                buf[e, sl] = buf[e, sl] * v
        return c2

    lax.fori_loop(0, _K // 16, group, 0)


def _seg_accum(sid, table, rows, cols, vals, idx_r, idx_c, idx_v,
               g0, g1, acc, sg0, sg1, ss0, ss1):
    nblk = _C0 // _BLK
    base = sid * _C0

    def block(b, carry):
        bb = base + b * _BLK
        pltpu.sync_copy(rows.at[pl.ds(bb, _BLK)], idx_r)
        pltpu.sync_copy(cols.at[pl.ds(bb, _BLK)], idx_c)
        pltpu.sync_copy(vals.at[pl.ds(bb, _BLK)], idx_v)
        pltpu.async_copy(table.at[idx_c.at[0]], g0, sg0)
        pltpu.async_copy(table.at[idx_c.at[1]], g1, sg1)

        def pair(j2, c2):
            j = 2 * j2
            pltpu.make_async_copy(table.at[idx_c.at[j]], g0, sg0).wait()
            _scale(idx_v, j, g0)
            sc0 = pltpu.async_copy(g0, acc.at[idx_r.at[j]], ss0, add=True)

            pltpu.make_async_copy(table.at[idx_c.at[j + 1]], g1, sg1).wait()
            _scale(idx_v, j + 1, g1)
            sc1 = pltpu.async_copy(g1, acc.at[idx_r.at[j + 1]], ss1, add=True)

            sc0.wait()

            @pl.when(j2 + 1 < _BLK // 2)
            def _():
                pltpu.async_copy(table.at[idx_c.at[j + 2]], g0, sg0)

            sc1.wait()

            @pl.when(j2 + 1 < _BLK // 2)
            def _():
                pltpu.async_copy(table.at[idx_c.at[j + 3]], g1, sg1)

            return c2

        lax.fori_loop(0, _BLK // 2, pair, 0)
        return carry

    lax.fori_loop(0, nblk, block, 0)


def _make_sc_spmm(n_seg):
    mesh = plsc.VectorSubcoreMesh(core_axis_name="c", subcore_axis_name="s")

    @functools.partial(
        pl.kernel,
        out_type=jax.ShapeDtypeStruct((_NP, _H), jnp.float32),
        mesh=mesh,
        scratch_types=[
            pltpu.VMEM((_BLK, _K), jnp.int32),
            pltpu.VMEM((_BLK, _K), jnp.int32),
            pltpu.VMEM((_BLK, _K), jnp.float32),
            pltpu.VMEM((_K, _H), jnp.float32),
            pltpu.VMEM((_K, _H), jnp.float32),
            pltpu.VMEM_SHARED((_NP, _H), jnp.float32),
            pltpu.SemaphoreType.DMA,
            pltpu.SemaphoreType.DMA,
            pltpu.SemaphoreType.DMA,
            pltpu.SemaphoreType.DMA,
        ],
    )
    def spmm(*refs):
        segs = refs[0:4 * n_seg]
        out = refs[4 * n_seg]
        (idx_r, idx_c, idx_v, g0, g1, acc,
         sg0, sg1, ss0, ss1) = refs[1 + 4 * n_seg:]

        cid = lax.axis_index("c")
        sid = lax.axis_index("s")

        @pl.when(cid == 0)
        def _body():
            def zrow(i, c2):
                for r in range(_H // 16):
                    g0[i, pl.ds(r * 16, 16)] = jnp.zeros((16,), jnp.float32)
                return c2

            lax.fori_loop(0, _K, zrow, 0)
            row0 = sid * _RPS
            for t in range(_RPS // _K):
                pltpu.sync_copy(g0, acc.at[pl.ds(row0 + t * _K, _K)])
            plsc.subcore_barrier()

            for s in range(n_seg):
                table, rows, cols, vals = segs[4 * s:4 * s + 4]
                _seg_accum(sid, table, rows, cols, vals, idx_r, idx_c,
                           idx_v, g0, g1, acc, sg0, sg1, ss0, ss1)

            plsc.subcore_barrier()
            pltpu.sync_copy(acc.at[pl.ds(row0, _RPS)],
                            out.at[pl.ds(row0, _RPS)])

    return spmm


_sc_spmm1 = _make_sc_spmm(1)
_sc_spmm2 = _make_sc_spmm(2)


_BR = 2000
_GRID = _N // _BR


def _mm(a, b):
    return jnp.dot(a, b, preferred_element_type=jnp.float32,
                   precision=lax.Precision.HIGHEST)


def _full(shape):
    return pl.BlockSpec(shape, lambda i: (0,) * len(shape))


def _rows(shape):
    return pl.BlockSpec(shape, lambda i: (i,) + (0,) * (len(shape) - 1))


def _enc_body(x0, tri, e0w, e0b, e2w, e2b, c0w, c2w,
              x0h_o, x2h_o, y0_o, y2_o):
    x0h = _swish(_mm(x0[...], e0w[...]) + e0b[...])
    x2h = _swish(_mm(tri[...], e2w[...]) + e2b[...])
    x0h_o[...] = x0h
    x2h_o[...] = x2h
    y0_o[...] = _mm(x0h, c0w[...])
    y2_o[...] = _mm(x2h, c2w[...])


def _encoder(x0, tri, e0w, e0b, e2w, e2b, c0w, c2w):
    n128 = jax.ShapeDtypeStruct((_N, _H), jnp.float32)
    return pl.pallas_call(
        _enc_body,
        grid=(_GRID,),
        in_specs=[_rows((_BR, _H)), _rows((_BR, 16)), _full((_H, _H)),
                  _full((1, _H)), _full((16, _H)), _full((1, _H)),
                  _full((_H, _H)), _full((_H, _H))],
        out_specs=[_rows((_BR, _H))] * 4,
        out_shape=[n128] * 4,
    )(x0, tri, e0w, e0b, e2w, e2b, c0w, c2w)


def _mid_body(p0, p2, c0w, x0h1_o, x2h1_o, y0_o):
    x0h1 = _swish(p0[...])
    x0h1_o[...] = x0h1
    x2h1_o[...] = _swish(p2[...])
    y0_o[...] = _mm(x0h1, c0w[...])


def _mid(p0, p2, c0w):
    n128 = jax.ShapeDtypeStruct((_N, _H), jnp.float32)
    pspec = pl.BlockSpec((_BR, _H), lambda i: (i, 0))
    return pl.pallas_call(
        _mid_body,
        grid=(_GRID,),
        in_specs=[pspec, pspec, _full((_H, _H))],
        out_specs=[_rows((_BR, _H))] * 3,
        out_shape=[n128] * 3,
    )(p0, p2, c0w)


def _fin_body(p0, x0h0, x0h1, pw0, pw1, pw2, pb, dw, db, out_o):
    x0h2 = _swish(p0[...])
    t = (_mm(x0h0[...], pw0[...]) + _mm(x0h1[...], pw1[...])
         + _mm(x0h2, pw2[...]) + pb[...])
    out_o[...] = _swish(_mm(t, dw[...]) + db[...])


def _final(p0, x0h0, x0h1, pw0, pw1, pw2, pb, dw, db):
    pspec = pl.BlockSpec((_BR, _H), lambda i: (i, 0))
    return pl.pallas_call(
        _fin_body,
        grid=(_GRID,),
        in_specs=[pspec, _rows((_BR, _H)), _rows((_BR, _H)),
                  _full((_H, _H)), _full((_H, _H)), _full((_H, _H)),
                  _full((1, _H)), _full((_H, _H)), _full((1, _H))],
        out_specs=_rows((_BR, _H)),
        out_shape=jax.ShapeDtypeStruct((_N, _H), jnp.float32),
    )(p0, x0h0, x0h1, pw0, pw1, pw2, pb, dw, db)


def kernel(x, pos, edge_attr, tri_attr, B1_rows, B1_cols, B1_vals,
           B2_rows, B2_cols, B2_vals, enc0_W, enc0_b, enc1_W, enc1_b,
           enc2_W, enc2_b, conv0_W, conv1_W, conv2_W, alpha, proj_W,
           proj_b, dec_W, dec_b):
    x0 = jnp.concatenate([x, pos], axis=-1)

    def _ed(a):
        return jnp.pad(a, (0, _EPAD - _NNZ)).reshape(-1, _K)

    r1 = _ed(B1_rows)
    c1 = _ed(B1_cols)
    r2 = _ed(B2_rows)
    c2 = _ed(B2_cols)
    va = _ed(alpha * B1_vals)
    vb = _ed((1.0 - alpha) * B2_vals)
    v2 = _ed(B2_vals)

    x0h0, x2h, y0, y2 = _encoder(
        x0, tri_attr, enc0_W, enc0_b.reshape(1, _H), enc2_W,
        enc2_b.reshape(1, _H), conv0_W, conv2_W)

    p0 = _sc_spmm2(y0, r1, c1, va, x2h, r2, c2, vb)
    p2 = _sc_spmm1(y2, r2, c2, v2)

    x0h1, x2h1, y0b = _mid(p0, p2, conv0_W)

    p0b = _sc_spmm2(y0b, r1, c1, va, x2h1, r2, c2, vb)

    return _final(p0b, x0h0, x0h1, proj_W[0:_H], proj_W[_H:2 * _H],
                  proj_W[2 * _H:3 * _H], proj_b.reshape(1, _H),
                  dec_W[:, :, 1].T, dec_b.reshape(1, _H))


# speedup vs baseline: 1.6602x; 1.6602x over previous
import functools

import jax
import jax.numpy as jnp
import numpy as np
from jax import lax
from jax.experimental import pallas as pl
from jax.experimental.pallas import tpu as pltpu
from jax.experimental.pallas import tpu_sc as plsc

_N = 10000
_NP = 10240
_H = 128
_NNZ = 320000
_NC = 2
_NS = 16
_NW = _NC * _NS
_K = 128
_C0 = 144
_C1 = 16
_NCHUNK = _NS * (_C0 + _C1)
_BLK = 16
_EPAD = _K * _NCHUNK
_RPS = _NP // _NS


def _swish(x):
    return x * jax.nn.sigmoid(x)


def _bcast_lane(vv, l):
    return lax.gather(
        vv, jnp.full((16, 1), l, jnp.int32),
        lax.GatherDimensionNumbers(
            offset_dims=(), collapsed_slice_dims=(0,), start_index_map=(0,)),
        (1,), mode=lax.GatherScatterMode.PROMISE_IN_BOUNDS)


def _scale(idx_v, j, buf):
    def group(g, c2):
        vv = idx_v[j, pl.ds(g * 16, 16)]
        for l in range(16):
            v = _bcast_lane(vv, l)
            e = g * 16 + l
            for r in range(_H // 16):
                sl = pl.ds(r * 16, 16)
                buf[e, sl] = buf[e, sl] * v
        return c2

    lax.fori_loop(0, _K // 16, group, 0)


def _seg_accum(cid, sid, table, rows, cols, vals, idx_r, idx_c, idx_v,
               g0, g1, acc, sg0, sg1, ss0, ss1):
    nblk = lax.select(cid == 0, _C0 // _BLK, _C1 // _BLK)
    base = lax.select(cid == 0, sid * _C0, _NS * _C0 + sid * _C1)

    def block(b, carry):
        bb = base + b * _BLK
        pltpu.sync_copy(rows.at[pl.ds(bb, _BLK)], idx_r)
        pltpu.sync_copy(cols.at[pl.ds(bb, _BLK)], idx_c)
        pltpu.sync_copy(vals.at[pl.ds(bb, _BLK)], idx_v)
        pltpu.async_copy(table.at[idx_c.at[0]], g0, sg0)
        pltpu.async_copy(table.at[idx_c.at[1]], g1, sg1)

        def pair(j2, c2):
            j = 2 * j2
            pltpu.make_async_copy(table.at[idx_c.at[j]], g0, sg0).wait()
            _scale(idx_v, j, g0)
            sc0 = pltpu.async_copy(g0, acc.at[idx_r.at[j]], ss0, add=True)

            pltpu.make_async_copy(table.at[idx_c.at[j + 1]], g1, sg1).wait()
            _scale(idx_v, j + 1, g1)
            sc1 = pltpu.async_copy(g1, acc.at[idx_r.at[j + 1]], ss1, add=True)

            sc0.wait()

            @pl.when(j2 + 1 < _BLK // 2)
            def _():
                pltpu.async_copy(table.at[idx_c.at[j + 2]], g0, sg0)

            sc1.wait()

            @pl.when(j2 + 1 < _BLK // 2)
            def _():
                pltpu.async_copy(table.at[idx_c.at[j + 3]], g1, sg1)

            return c2

        lax.fori_loop(0, _BLK // 2, pair, 0)
        return carry

    lax.fori_loop(0, nblk, block, 0)


def _make_sc_spmm(n_seg):
    mesh = plsc.VectorSubcoreMesh(core_axis_name="c", subcore_axis_name="s")

    @functools.partial(
        pl.kernel,
        out_type=jax.ShapeDtypeStruct((_NC, _NP, _H), jnp.float32),
        mesh=mesh,
        scratch_types=[
            pltpu.VMEM((_BLK, _K), jnp.int32),
            pltpu.VMEM((_BLK, _K), jnp.int32),
            pltpu.VMEM((_BLK, _K), jnp.float32),
            pltpu.VMEM((_K, _H), jnp.float32),
            pltpu.VMEM((_K, _H), jnp.float32),
            pltpu.VMEM_SHARED((_NP, _H), jnp.float32),
            pltpu.SemaphoreType.DMA,
            pltpu.SemaphoreType.DMA,
            pltpu.SemaphoreType.DMA,
            pltpu.SemaphoreType.DMA,
        ],
    )
    def spmm(*refs):
        segs = refs[0:4 * n_seg]
        out = refs[4 * n_seg]
        (idx_r, idx_c, idx_v, g0, g1, acc,
         sg0, sg1, ss0, ss1) = refs[1 + 4 * n_seg:]

        cid = lax.axis_index("c")
        sid = lax.axis_index("s")

        def zrow(i, c2):
            for r in range(_H // 16):
                g0[i, pl.ds(r * 16, 16)] = jnp.zeros((16,), jnp.float32)
            return c2

        lax.fori_loop(0, _K, zrow, 0)
        row0 = sid * _RPS
        for t in range(_RPS // _K):
            pltpu.sync_copy(g0, acc.at[pl.ds(row0 + t * _K, _K)])
        plsc.subcore_barrier()

        for s in range(n_seg):
            table, rows, cols, vals = segs[4 * s:4 * s + 4]
            _seg_accum(cid, sid, table, rows, cols, vals, idx_r, idx_c,
                       idx_v, g0, g1, acc, sg0, sg1, ss0, ss1)

        plsc.subcore_barrier()
        pltpu.sync_copy(acc.at[pl.ds(row0, _RPS)],
                        out.at[cid, pl.ds(row0, _RPS)])

    return spmm


_sc_spmm1 = _make_sc_spmm(1)
_sc_spmm2 = _make_sc_spmm(2)


_BR = 2000
_GRID = _N // _BR


def _mm(a, b):
    return jnp.dot(a, b, preferred_element_type=jnp.float32,
                   precision=lax.Precision.HIGHEST)


def _full(shape):
    return pl.BlockSpec(shape, lambda i: (0,) * len(shape))


def _rows(shape):
    return pl.BlockSpec(shape, lambda i: (i,) + (0,) * (len(shape) - 1))


def _enc_body(x0, tri, e0w, e0b, e2w, e2b, c0w, c2w,
              x0h_o, x2h_o, y0_o, y2_o):
    x0h = _swish(_mm(x0[...], e0w[...]) + e0b[...])
    x2h = _swish(_mm(tri[...], e2w[...]) + e2b[...])
    x0h_o[...] = x0h
    x2h_o[...] = x2h
    y0_o[...] = _mm(x0h, c0w[...])
    y2_o[...] = _mm(x2h, c2w[...])


def _encoder(x0, tri, e0w, e0b, e2w, e2b, c0w, c2w):
    n128 = jax.ShapeDtypeStruct((_N, _H), jnp.float32)
    return pl.pallas_call(
        _enc_body,
        grid=(_GRID,),
        in_specs=[_rows((_BR, _H)), _rows((_BR, 16)), _full((_H, _H)),
                  _full((1, _H)), _full((16, _H)), _full((1, _H)),
                  _full((_H, _H)), _full((_H, _H))],
        out_specs=[_rows((_BR, _H))] * 4,
        out_shape=[n128] * 4,
    )(x0, tri, e0w, e0b, e2w, e2b, c0w, c2w)


def _mid_body(p0, p2, c0w, x0h1_o, x2h1_o, y0_o):
    x0h1 = _swish(p0[0] + p0[1])
    x0h1_o[...] = x0h1
    x2h1_o[...] = _swish(p2[0] + p2[1])
    y0_o[...] = _mm(x0h1, c0w[...])


def _mid(p0, p2, c0w):
    n128 = jax.ShapeDtypeStruct((_N, _H), jnp.float32)
    pspec = pl.BlockSpec((_NC, _BR, _H), lambda i: (0, i, 0))
    return pl.pallas_call(
        _mid_body,
        grid=(_GRID,),
        in_specs=[pspec, pspec, _full((_H, _H))],
        out_specs=[_rows((_BR, _H))] * 3,
        out_shape=[n128] * 3,
    )(p0, p2, c0w)


def _fin_body(p0, x0h0, x0h1, pw0, pw1, pw2, pb, dw, db, out_o):
    x0h2 = _swish(p0[0] + p0[1])
    t = (_mm(x0h0[...], pw0[...]) + _mm(x0h1[...], pw1[...])
         + _mm(x0h2, pw2[...]) + pb[...])
    out_o[...] = _swish(_mm(t, dw[...]) + db[...])


def _final(p0, x0h0, x0h1, pw0, pw1, pw2, pb, dw, db):
    pspec = pl.BlockSpec((_NC, _BR, _H), lambda i: (0, i, 0))
    return pl.pallas_call(
        _fin_body,
        grid=(_GRID,),
        in_specs=[pspec, _rows((_BR, _H)), _rows((_BR, _H)),
                  _full((_H, _H)), _full((_H, _H)), _full((_H, _H)),
                  _full((1, _H)), _full((_H, _H)), _full((1, _H))],
        out_specs=_rows((_BR, _H)),
        out_shape=jax.ShapeDtypeStruct((_N, _H), jnp.float32),
    )(p0, x0h0, x0h1, pw0, pw1, pw2, pb, dw, db)


def kernel(x, pos, edge_attr, tri_attr, B1_rows, B1_cols, B1_vals,
           B2_rows, B2_cols, B2_vals, enc0_W, enc0_b, enc1_W, enc1_b,
           enc2_W, enc2_b, conv0_W, conv1_W, conv2_W, alpha, proj_W,
           proj_b, dec_W, dec_b):
    x0 = jnp.concatenate([x, pos], axis=-1)

    def _ed(a):
        return jnp.pad(a, (0, _EPAD - _NNZ)).reshape(-1, _K)

    r1 = _ed(B1_rows)
    c1 = _ed(B1_cols)
    r2 = _ed(B2_rows)
    c2 = _ed(B2_cols)
    va = _ed(alpha * B1_vals)
    vb = _ed((1.0 - alpha) * B2_vals)
    v2 = _ed(B2_vals)

    x0h0, x2h, y0, y2 = _encoder(
        x0, tri_attr, enc0_W, enc0_b.reshape(1, _H), enc2_W,
        enc2_b.reshape(1, _H), conv0_W, conv2_W)

    p0 = _sc_spmm2(y0, r1, c1, va, x2h, r2, c2, vb)
    p2 = _sc_spmm1(y2, r2, c2, v2)

    x0h1, x2h1, y0b = _mid(p0, p2, conv0_W)

    p0b = _sc_spmm2(y0b, r1, c1, va, x2h1, r2, c2, vb)

    return _final(p0b, x0h0, x0h1, proj_W[0:_H], proj_W[_H:2 * _H],
                  proj_W[2 * _H:3 * _H], proj_b.reshape(1, _H),
                  dec_W[:, :, 1].T, dec_b.reshape(1, _H))


# speedup vs baseline: 1.6608x; 1.0004x over previous
import functools

import jax
import jax.numpy as jnp
from jax import lax
from jax.experimental import pallas as pl
from jax.experimental.pallas import tpu as pltpu
from jax.experimental.pallas import tpu_sc as plsc

_N = 10000
_NP = 10240
_H = 128
_NNZ = 320000
_NC = 2
_NS = 16
_NW = _NC * _NS
_K = 128
_C0 = 144
_C1 = 16
_NCHUNK = _NS * (_C0 + _C1)
_BLK = 16
_EPAD = _K * _NCHUNK
_RPS = _NP // _NS


def _swish(x):
    return x * jax.nn.sigmoid(x)


def _bcast_lane(vv, l):
    return lax.gather(
        vv, jnp.full((16, 1), l, jnp.int32),
        lax.GatherDimensionNumbers(
            offset_dims=(), collapsed_slice_dims=(0,), start_index_map=(0,)),
        (1,), mode=lax.GatherScatterMode.PROMISE_IN_BOUNDS)


def _scale(idx_v, j, buf):
    def group(g, c2):
        vv = idx_v[j, pl.ds(g * 16, 16)]
        for l in range(16):
            v = _bcast_lane(vv, l)
            e = g * 16 + l
            for r in range(_H // 16):
                sl = pl.ds(r * 16, 16)
                buf[e, sl] = buf[e, sl] * v
        return c2

    lax.fori_loop(0, _K // 16, group, 0)


def _seg_accum(cid, sid, table, rows, cols, vals, idx_r, idx_c, idx_v,
               g0, g1, acc, sg0, sg1, ss0, ss1):
    nblk = lax.select(cid == 0, _C0 // _BLK, _C1 // _BLK)
    base = lax.select(cid == 0, sid * _C0, _NS * _C0 + sid * _C1)

    def block(b, carry):
        bb = base + b * _BLK
        pltpu.sync_copy(rows.at[pl.ds(bb, _BLK)], idx_r)
        pltpu.sync_copy(cols.at[pl.ds(bb, _BLK)], idx_c)
        pltpu.sync_copy(vals.at[pl.ds(bb, _BLK)], idx_v)
        pltpu.async_copy(table.at[idx_c.at[0]], g0, sg0)
        pltpu.async_copy(table.at[idx_c.at[1]], g1, sg1)

        def pair(j2, c2):
            j = 2 * j2
            pltpu.make_async_copy(table.at[idx_c.at[j]], g0, sg0).wait()
            _scale(idx_v, j, g0)
            sc0 = pltpu.async_copy(g0, acc.at[idx_r.at[j]], ss0, add=True)

            pltpu.make_async_copy(table.at[idx_c.at[j + 1]], g1, sg1).wait()
            _scale(idx_v, j + 1, g1)
            sc1 = pltpu.async_copy(g1, acc.at[idx_r.at[j + 1]], ss1, add=True)

            sc0.wait()

            @pl.when(j2 + 1 < _BLK // 2)
            def _():
                pltpu.async_copy(table.at[idx_c.at[j + 2]], g0, sg0)

            sc1.wait()

            @pl.when(j2 + 1 < _BLK // 2)
            def _():
                pltpu.async_copy(table.at[idx_c.at[j + 3]], g1, sg1)

            return c2

        lax.fori_loop(0, _BLK // 2, pair, 0)
        return carry

    lax.fori_loop(0, nblk, block, 0)


def _make_sc_spmm(n_seg):
    mesh = plsc.VectorSubcoreMesh(core_axis_name="c", subcore_axis_name="s")

    @functools.partial(
        pl.kernel,
        out_type=jax.ShapeDtypeStruct((_NC, _NP, _H), jnp.float32),
        mesh=mesh,
        scratch_types=[
            pltpu.VMEM((_BLK, _K), jnp.int32),
            pltpu.VMEM((_BLK, _K), jnp.int32),
            pltpu.VMEM((_BLK, _K), jnp.float32),
            pltpu.VMEM((_K, _H), jnp.float32),
            pltpu.VMEM((_K, _H), jnp.float32),
            pltpu.VMEM_SHARED((_NP, _H), jnp.float32),
            pltpu.SemaphoreType.DMA,
            pltpu.SemaphoreType.DMA,
            pltpu.SemaphoreType.DMA,
            pltpu.SemaphoreType.DMA,
        ],
    )
    def spmm(*refs):
        segs = refs[0:4 * n_seg]
        out = refs[4 * n_seg]
        (idx_r, idx_c, idx_v, g0, g1, acc,
         sg0, sg1, ss0, ss1) = refs[1 + 4 * n_seg:]

        cid = lax.axis_index("c")
        sid = lax.axis_index("s")

        def zrow(i, c2):
            for r in range(_H // 16):
                g0[i, pl.ds(r * 16, 16)] = jnp.zeros((16,), jnp.float32)
            return c2

        lax.fori_loop(0, _K, zrow, 0)
        row0 = sid * _RPS
        for t in range(_RPS // _K):
            pltpu.sync_copy(g0, acc.at[pl.ds(row0 + t * _K, _K)])
        plsc.subcore_barrier()

        for s in range(n_seg):
            table, rows, cols, vals = segs[4 * s:4 * s + 4]
            _seg_accum(cid, sid, table, rows, cols, vals, idx_r, idx_c,
                       idx_v, g0, g1, acc, sg0, sg1, ss0, ss1)

        plsc.subcore_barrier()
        pltpu.sync_copy(acc.at[pl.ds(row0, _RPS)],
                        out.at[cid, pl.ds(row0, _RPS)])

    return spmm


_sc_spmm1 = _make_sc_spmm(1)
_sc_spmm2 = _make_sc_spmm(2)


_BR = 2000
_GRID = _N // _BR


def _mm(a, b):
    return jnp.dot(a, b, preferred_element_type=jnp.float32,
                   precision=lax.Precision.HIGHEST)


def _full(shape):
    return pl.BlockSpec(shape, lambda i: (0,) * len(shape))


def _rows(shape):
    return pl.BlockSpec(shape, lambda i: (i,) + (0,) * (len(shape) - 1))


def _enc_body(x0, tri, e0w, e0b, e2w, e2b, c0w, c2w,
              x0h_o, x2h_o, y0_o, y2_o):
    x0h = _swish(_mm(x0[...], e0w[...]) + e0b[...])
    x2h = _swish(_mm(tri[...], e2w[...]) + e2b[...])
    x0h_o[...] = x0h
    x2h_o[...] = x2h
    y0_o[...] = _mm(x0h, c0w[...])
    y2_o[...] = _mm(x2h, c2w[...])


def _encoder(x0, tri, e0w, e0b, e2w, e2b, c0w, c2w):
    n128 = jax.ShapeDtypeStruct((_N, _H), jnp.float32)
    return pl.pallas_call(
        _enc_body,
        grid=(_GRID,),
        in_specs=[_rows((_BR, _H)), _rows((_BR, 16)), _full((_H, _H)),
                  _full((1, _H)), _full((16, _H)), _full((1, _H)),
                  _full((_H, _H)), _full((_H, _H))],
        out_specs=[_rows((_BR, _H))] * 4,
        out_shape=[n128] * 4,
    )(x0, tri, e0w, e0b, e2w, e2b, c0w, c2w)


def _mid_body(p0, p2, c0w, x0h1_o, x2h1_o, y0_o):
    x0h1 = _swish(p0[0] + p0[1])
    x0h1_o[...] = x0h1
    x2h1_o[...] = _swish(p2[0] + p2[1])
    y0_o[...] = _mm(x0h1, c0w[...])


def _mid(p0, p2, c0w):
    n128 = jax.ShapeDtypeStruct((_N, _H), jnp.float32)
    pspec = pl.BlockSpec((_NC, _BR, _H), lambda i: (0, i, 0))
    return pl.pallas_call(
        _mid_body,
        grid=(_GRID,),
        in_specs=[pspec, pspec, _full((_H, _H))],
        out_specs=[_rows((_BR, _H))] * 3,
        out_shape=[n128] * 3,
    )(p0, p2, c0w)


def _fin_body(p0, x0h0, x0h1, pw0, pw1, pw2, pb, dw, db, out_o):
    x0h2 = _swish(p0[0] + p0[1])
    t = (_mm(x0h0[...], pw0[...]) + _mm(x0h1[...], pw1[...])
         + _mm(x0h2, pw2[...]) + pb[...])
    out_o[...] = _swish(_mm(t, dw[...]) + db[...])


def _final(p0, x0h0, x0h1, pw0, pw1, pw2, pb, dw, db):
    pspec = pl.BlockSpec((_NC, _BR, _H), lambda i: (0, i, 0))
    return pl.pallas_call(
        _fin_body,
        grid=(_GRID,),
        in_specs=[pspec, _rows((_BR, _H)), _rows((_BR, _H)),
                  _full((_H, _H)), _full((_H, _H)), _full((_H, _H)),
                  _full((1, _H)), _full((_H, _H)), _full((1, _H))],
        out_specs=_rows((_BR, _H)),
        out_shape=jax.ShapeDtypeStruct((_N, _H), jnp.float32),
    )(p0, x0h0, x0h1, pw0, pw1, pw2, pb, dw, db)


def kernel(x, pos, edge_attr, tri_attr, B1_rows, B1_cols, B1_vals,
           B2_rows, B2_cols, B2_vals, enc0_W, enc0_b, enc1_W, enc1_b,
           enc2_W, enc2_b, conv0_W, conv1_W, conv2_W, alpha, proj_W,
           proj_b, dec_W, dec_b):
    x0 = jnp.concatenate([x, pos], axis=-1)

    def _ed(a):
        return jnp.pad(a, (0, _EPAD - _NNZ)).reshape(-1, _K)

    r1 = _ed(B1_rows)
    c1 = _ed(B1_cols)
    r2 = _ed(B2_rows)
    c2 = _ed(B2_cols)
    va = _ed(alpha * B1_vals)
    vb = _ed((1.0 - alpha) * B2_vals)
    v2 = _ed(B2_vals)

    x0h0, x2h, y0, y2 = _encoder(
        x0, tri_attr, enc0_W, enc0_b.reshape(1, _H), enc2_W,
        enc2_b.reshape(1, _H), conv0_W, conv2_W)

    p0 = _sc_spmm2(y0, r1, c1, va, x2h, r2, c2, vb)
    p2 = _sc_spmm1(y2, r2, c2, v2)

    x0h1, x2h1, y0b = _mid(p0, p2, conv0_W)

    p0b = _sc_spmm2(y0b, r1, c1, va, x2h1, r2, c2, vb)

    return _final(p0b, x0h0, x0h1, proj_W[0:_H], proj_W[_H:2 * _H],
                  proj_W[2 * _H:3 * _H], proj_b.reshape(1, _H),
                  dec_W[:, :, 1].T, dec_b.reshape(1, _H))
